# Initial kernel scaffold; baseline (speedup 1.0000x reference)
import functools

import jax
import jax.numpy as jnp
from jax import lax
from jax.experimental import pallas as pl
from jax.experimental.pallas import tpu as pltpu
from jax.experimental.pallas import tpu_sc as plsc

NC = 2
NS = 16
NW = NC * NS
LANES = 16

_MESH = plsc.VectorSubcoreMesh(core_axis_name="c", subcore_axis_name="s")


def _sc_degree(row, n):
    e = row.shape[0]
    epw = e // NW
    b = 80
    nrounds = epw // b
    nps = n // NS
    zr = 125
    assert nrounds * b == epw and nps % zr == 0

    @functools.partial(
        pl.kernel,
        out_type=jax.ShapeDtypeStruct((NC, n, LANES), jnp.float32),
        mesh=_MESH,
        scratch_types=[
            pltpu.VMEM((b,), jnp.int32),
            pltpu.VMEM((b, LANES), jnp.float32),
            pltpu.VMEM((zr, LANES), jnp.float32),
            pltpu.VMEM_SHARED((n, LANES), jnp.float32),
        ],
    )
    def k(row_hbm, out_hbm, idx_v, ones_v, z_v, acc_sh):
        cid = lax.axis_index("c")
        sid = lax.axis_index("s")
        wid = cid * NS + sid

        @pl.loop(0, b)
        def _(i):
            ones_v[i, :] = jnp.ones((LANES,), jnp.float32)

        @pl.loop(0, zr)
        def _(i):
            z_v[i, :] = jnp.zeros((LANES,), jnp.float32)

        @pl.loop(0, nps // zr)
        def _(t):
            pltpu.sync_copy(z_v, acc_sh.at[pl.ds(sid * nps + t * zr, zr)])

        plsc.subcore_barrier()

        base = wid * epw

        @pl.loop(0, nrounds)
        def _(r):
            pltpu.sync_copy(row_hbm.at[pl.ds(base + r * b, b)], idx_v)
            pltpu.sync_copy(ones_v, acc_sh.at[idx_v], add=True)

        plsc.subcore_barrier()
        pltpu.sync_copy(
            acc_sh.at[pl.ds(sid * nps, nps)],
            out_hbm.at[cid, pl.ds(sid * nps, nps)],
        )

    return k(row)


def _sc_norm(row, col, dinv):
    e = row.shape[0]
    n = dinv.shape[0]
    epw = e // NW
    b = 80
    nrounds = epw // b

    @functools.partial(
        pl.kernel,
        out_type=jax.ShapeDtypeStruct((e,), jnp.float32),
        mesh=_MESH,
        scratch_types=[
            pltpu.VMEM((n,), jnp.float32),
            pltpu.VMEM((b,), jnp.int32),
            pltpu.VMEM((b,), jnp.int32),
            pltpu.VMEM((b,), jnp.float32),
        ],
    )
    def k(row_hbm, col_hbm, dinv_hbm, out_hbm, dinv_v, row_v, col_v, norm_v):
        cid = lax.axis_index("c")
        sid = lax.axis_index("s")
        wid = cid * NS + sid
        base = wid * epw
        pltpu.sync_copy(dinv_hbm, dinv_v)

        @pl.loop(0, nrounds)
        def _(r):
            pltpu.sync_copy(row_hbm.at[pl.ds(base + r * b, b)], row_v)
            pltpu.sync_copy(col_hbm.at[pl.ds(base + r * b, b)], col_v)

            @pl.loop(0, b // LANES)
            def _(j):
                ir = row_v[pl.ds(j * LANES, LANES)]
                ic = col_v[pl.ds(j * LANES, LANES)]
                gr = plsc.load_gather(dinv_v, [ir])
                gc = plsc.load_gather(dinv_v, [ic])
                norm_v[pl.ds(j * LANES, LANES)] = gr * gc

            pltpu.sync_copy(norm_v, out_hbm.at[pl.ds(base + r * b, b)])

    return k(row, col, dinv)


def _sc_message(h_lin, ee3, l, row, col, norm, n, d):
    e = row.shape[0]
    epw = e // NW
    b = 80
    nrounds = epw // b
    nps = n // NS
    zr = 125
    nchunks = d // LANES
    assert nrounds * b == epw and (nps % zr) == 0

    @functools.partial(
        pl.kernel,
        out_type=jax.ShapeDtypeStruct((NC, n, d), jnp.float32),
        mesh=_MESH,
        scratch_types=[
            pltpu.VMEM((b,), jnp.int32),
            pltpu.VMEM((b,), jnp.int32),
            pltpu.VMEM((b,), jnp.float32),
            pltpu.VMEM((b, d), jnp.float32),
            pltpu.VMEM((b, d), jnp.float32),
            pltpu.VMEM((zr, d), jnp.float32),
            pltpu.VMEM_SHARED((n, d), jnp.float32),
        ],
    )
    def k(h_hbm, ee_hbm, row_hbm, col_hbm, norm_hbm, out_hbm,
          row_v, col_v, norm_v, m_v, ee_v, z_v, acc_sh):
        cid = lax.axis_index("c")
        sid = lax.axis_index("s")
        wid = cid * NS + sid
        base = wid * epw

        @pl.loop(0, zr)
        def _(i):
            for c in range(nchunks):
                z_v[i, pl.ds(c * LANES, LANES)] = jnp.zeros((LANES,), jnp.float32)

        @pl.loop(0, nps // zr)
        def _(t):
            pltpu.sync_copy(z_v, acc_sh.at[pl.ds(sid * nps + t * zr, zr)])

        plsc.subcore_barrier()

        @pl.loop(0, nrounds)
        def _(r):
            e0 = base + r * b
            pltpu.sync_copy(row_hbm.at[pl.ds(e0, b)], row_v)
            pltpu.sync_copy(col_hbm.at[pl.ds(e0, b)], col_v)
            pltpu.sync_copy(norm_hbm.at[pl.ds(e0, b)], norm_v)
            pltpu.sync_copy(ee_hbm.at[l, pl.ds(e0, b)], ee_v)
            pltpu.sync_copy(h_hbm.at[row_v], m_v)

            @pl.loop(0, b)
            def _(j):
                nj = norm_v[j]
                for c in range(nchunks):
                    sl = pl.ds(c * LANES, LANES)
                    v = m_v[j, sl] + ee_v[j, sl]
                    m_v[j, sl] = jnp.maximum(v, 0.0) * nj

            pltpu.sync_copy(m_v, acc_sh.at[col_v], add=True)

        plsc.subcore_barrier()
        pltpu.sync_copy(
            acc_sh.at[pl.ds(sid * nps, nps)],
            out_hbm.at[cid, pl.ds(sid * nps, nps)],
        )

    return k(h_lin, ee3, row, col, norm)


def _tc_deg_finish(parts):
    n = parts.shape[1]

    def body(p_ref, dinv_ref, dinv2_ref):
        deg = p_ref[0, :, 0:1] + p_ref[1, :, 0:1] + 1.0
        dinv_ref[...] = lax.rsqrt(deg)
        dinv2_ref[...] = 1.0 / deg

    return pl.pallas_call(
        body,
        out_shape=[
            jax.ShapeDtypeStruct((n, 1), jnp.float32),
            jax.ShapeDtypeStruct((n, 1), jnp.float32),
        ],
    )(parts)


def _tc_ee(edge_attr, Wes, bes):
    e, de = edge_attr.shape
    nl, _, d = Wes.shape
    be_blk = 2000
    grid = (nl, e // be_blk)

    def body(ea_ref, w_ref, b_ref, out_ref):
        acc = jnp.dot(ea_ref[...], w_ref[0], preferred_element_type=jnp.float32)
        out_ref[0] = acc + b_ref[0]

    return pl.pallas_call(
        body,
        grid=grid,
        in_specs=[
            pl.BlockSpec((be_blk, de), lambda l, i: (i, 0)),
            pl.BlockSpec((1, de, d), lambda l, i: (l, 0, 0)),
            pl.BlockSpec((1, d), lambda l, i: (l, 0)),
        ],
        out_specs=pl.BlockSpec((1, be_blk, d), lambda l, i: (l, i, 0)),
        out_shape=jax.ShapeDtypeStruct((nl, e, d), jnp.float32),
    )(edge_attr, Wes, bes)


def _tc_matmul(h, W, b):
    n, d = h.shape

    def body(h_ref, w_ref, b_ref, out_ref):
        out_ref[...] = (
            jnp.dot(h_ref[...], w_ref[...], preferred_element_type=jnp.float32)
            + b_ref[...]
        )

    return pl.pallas_call(
        body,
        out_shape=jax.ShapeDtypeStruct((n, d), jnp.float32),
    )(h, W, b.reshape(1, d))


def _tc_combine(parts, h_lin, root, dinv2):
    n, d = h_lin.shape

    def body(p_ref, h_ref, r_ref, d2_ref, out_ref):
        aggr = p_ref[0] + p_ref[1]
        self_term = jnp.maximum(h_ref[...] + r_ref[...], 0.0) * d2_ref[...]
        out_ref[...] = jnp.maximum(aggr + self_term, 0.0)

    return pl.pallas_call(
        body,
        out_shape=jax.ShapeDtypeStruct((n, d), jnp.float32),
    )(parts, h_lin, root.reshape(1, d), dinv2)


def _tc_final(hs, gamma, beta, Wout, bout):
    n, d = hs[0].shape
    nseg = len(hs)

    def body(x0, x1, x2, x3, g_ref, be_ref, w_ref, bo_ref, out_ref):
        acc = jnp.zeros((n, d), jnp.float32)
        for i, xr in enumerate((x0, x1, x2, x3)):
            xv = xr[...]
            m = jnp.mean(xv, axis=0, keepdims=True)
            ctr = xv - m
            var = jnp.mean(ctr * ctr, axis=0, keepdims=True)
            xn = ctr * lax.rsqrt(var + 1e-5) * g_ref[i] + be_ref[i]
            acc = acc + jnp.dot(xn, w_ref[i], preferred_element_type=jnp.float32)
        out_ref[...] = acc + bo_ref[...]

    return pl.pallas_call(
        body,
        out_shape=jax.ShapeDtypeStruct((n, d), jnp.float32),
    )(*hs, gamma.reshape(nseg, d), beta.reshape(nseg, d),
      Wout.reshape(nseg, d, d), bout.reshape(1, d))


def kernel(x, edge_index, edge_attr, Ws, bs, Wes, bes, roots, gamma, beta,
           Wout, bout):
    n, d = x.shape
    nl = Ws.shape[0]
    row = edge_index[0]
    col = edge_index[1]

    deg_parts = _sc_degree(row, n)
    dinv, dinv2 = _tc_deg_finish(deg_parts)
    norm = _sc_norm(row, col, dinv.reshape(n))
    ee3 = _tc_ee(edge_attr, Wes, bes)

    h = x
    hs = [x]
    for l in range(nl):
        h_lin = _tc_matmul(h, Ws[l], bs[l])
        parts = _sc_message(h_lin, ee3, l, row, col, norm, n, d)
        h = _tc_combine(parts, h_lin, roots[l], dinv2)
        hs.append(h)

    return _tc_final(hs, gamma, beta, Wout, bout)


# speedup vs baseline: 3.2802x; 3.2802x over previous
import dataclasses
import functools

import jax
import jax.numpy as jnp
from jax import lax
from jax.experimental import pallas as pl
from jax.experimental.pallas import tpu as pltpu
from jax.experimental.pallas import tpu_sc as plsc

NC = 2
NS = 16
NW = NC * NS
LANES = 16

_MESH = plsc.VectorSubcoreMesh(
    core_axis_name="c", subcore_axis_name="s", num_cores=NC, num_subcores=NS
)

_SC_PARAMS = pltpu.CompilerParams()
if "needs_layout_passes" in pltpu.CompilerParams.__dataclass_fields__:
    _SC_PARAMS = dataclasses.replace(_SC_PARAMS, needs_layout_passes=False)

_SC_LINEAR = dataclasses.replace(
    pltpu.CompilerParams(), use_tc_tiling_on_sc=False
)


def _scatter_add(src_ref, dst_ref, idx_ref):
    pltpu.sync_copy(src_ref, dst_ref.at[idx_ref], add=True)


def _sc_degree(row, n):
    e = row.shape[0]
    epw = e // NW
    b = 80
    nrounds = epw // b
    nps = n // NS
    zr = 125
    assert nrounds * b == epw and nps % zr == 0

    @functools.partial(
        pl.kernel,
        out_type=jax.ShapeDtypeStruct((NC, NS, nps, LANES), jnp.float32),
        mesh=_MESH,
        compiler_params=_SC_LINEAR,
        scratch_types=[
            pltpu.VMEM((b,), jnp.int32),
            pltpu.VMEM((b, LANES), jnp.float32),
            pltpu.VMEM((zr, LANES), jnp.float32),
            pltpu.VMEM_SHARED((n, LANES), jnp.float32),
        ],
    )
    def k(row_hbm, out_hbm, idx_v, ones_v, z_v, acc_sh):
        cid = lax.axis_index("c")
        sid = lax.axis_index("s")
        wid = cid * NS + sid

        @pl.loop(0, b)
        def _(i):
            ones_v[i, :] = jnp.ones((LANES,), jnp.float32)

        @pl.loop(0, zr)
        def _(i):
            z_v[i, :] = jnp.zeros((LANES,), jnp.float32)

        @pl.loop(0, nps // zr)
        def _(t):
            pltpu.sync_copy(z_v, acc_sh.at[pl.ds(sid * nps + t * zr, zr)])

        plsc.subcore_barrier()

        base = wid * epw

        @pl.loop(0, nrounds)
        def _(r):
            pltpu.sync_copy(row_hbm.at[pl.ds(base + r * b, b)], idx_v)
            _scatter_add(ones_v, acc_sh, idx_v)

        plsc.subcore_barrier()
        pltpu.sync_copy(
            acc_sh.at[pl.ds(sid * nps, nps)],
            out_hbm.at[cid, sid],
        )

    return k(row)


def _sc_norm(row, col, dinv):
    e = row.shape[0]
    n = dinv.shape[0]
    epw = e // NW
    b = 80
    nrounds = epw // b

    @functools.partial(
        pl.kernel,
        out_type=jax.ShapeDtypeStruct((e,), jnp.float32),
        mesh=_MESH,
        compiler_params=_SC_PARAMS,
        scratch_types=[
            pltpu.VMEM((n,), jnp.float32),
            pltpu.VMEM((b,), jnp.int32),
            pltpu.VMEM((b,), jnp.int32),
            pltpu.VMEM((b,), jnp.float32),
        ],
    )
    def k(row_hbm, col_hbm, dinv_hbm, out_hbm, dinv_v, row_v, col_v, norm_v):
        cid = lax.axis_index("c")
        sid = lax.axis_index("s")
        wid = cid * NS + sid
        base = wid * epw
        pltpu.sync_copy(dinv_hbm, dinv_v)

        @pl.loop(0, nrounds)
        def _(r):
            pltpu.sync_copy(row_hbm.at[pl.ds(base + r * b, b)], row_v)
            pltpu.sync_copy(col_hbm.at[pl.ds(base + r * b, b)], col_v)

            @pl.loop(0, b // LANES)
            def _(j):
                ir = row_v[pl.ds(j * LANES, LANES)]
                ic = col_v[pl.ds(j * LANES, LANES)]
                gr = plsc.load_gather(dinv_v, [ir])
                gc = plsc.load_gather(dinv_v, [ic])
                norm_v[pl.ds(j * LANES, LANES)] = gr * gc

            pltpu.sync_copy(norm_v, out_hbm.at[pl.ds(base + r * b, b)])

    return k(row, col, dinv)


def _sc_message(h_lin, ee3, l, row, col, norm, n, d):
    e = row.shape[0]
    epw = e // NW
    b = 80
    nrounds = epw // b
    nps = n // NS
    zr = 125
    nchunks = d // LANES
    assert nrounds * b == epw and (nps % zr) == 0

    @functools.partial(
        pl.kernel,
        out_type=jax.ShapeDtypeStruct((NC, NS, nps, d), jnp.float32),
        mesh=_MESH,
        scratch_types=[
            pltpu.VMEM((b,), jnp.int32),
            pltpu.VMEM((b,), jnp.int32),
            pltpu.VMEM((b,), jnp.float32),
            pltpu.VMEM((b, d), jnp.float32),
            pltpu.VMEM((b, d), jnp.float32),
            pltpu.VMEM((zr, d), jnp.float32),
            pltpu.VMEM_SHARED((n, d), jnp.float32),
        ],
    )
    def k(h_hbm, ee_hbm, row_hbm, col_hbm, norm_hbm, out_hbm,
          row_v, col_v, norm_v, m_v, ee_v, z_v, acc_sh):
        cid = lax.axis_index("c")
        sid = lax.axis_index("s")
        wid = cid * NS + sid
        base = wid * epw

        @pl.loop(0, zr)
        def _(i):
            for c in range(nchunks):
                z_v[i, pl.ds(c * LANES, LANES)] = jnp.zeros((LANES,), jnp.float32)

        @pl.loop(0, nps // zr)
        def _(t):
            pltpu.sync_copy(z_v, acc_sh.at[pl.ds(sid * nps + t * zr, zr)])

        plsc.subcore_barrier()

        @pl.loop(0, nrounds)
        def _(r):
            e0 = base + r * b
            pltpu.sync_copy(row_hbm.at[pl.ds(e0, b)], row_v)
            pltpu.sync_copy(col_hbm.at[pl.ds(e0, b)], col_v)
            pltpu.sync_copy(norm_hbm.at[pl.ds(e0, b)], norm_v)
            pltpu.sync_copy(ee_hbm.at[l, pl.ds(e0, b)], ee_v)
            pltpu.sync_copy(h_hbm.at[row_v], m_v)

            @pl.loop(0, b // LANES)
            def _(g):
                nv = norm_v[pl.ds(g * LANES, LANES)]
                for jj in range(LANES):
                    j = g * LANES + jj
                    nj = nv[jj]
                    for c in range(nchunks):
                        sl = pl.ds(c * LANES, LANES)
                        v = m_v[j, sl] + ee_v[j, sl]
                        m_v[j, sl] = jnp.maximum(v, 0.0) * nj

            _scatter_add(m_v, acc_sh, col_v)

        plsc.subcore_barrier()
        pltpu.sync_copy(
            acc_sh.at[pl.ds(sid * nps, nps)],
            out_hbm.at[cid, sid],
        )

    return k(h_lin, ee3, row, col, norm)


def _tc_deg_finish(parts):
    n = parts.shape[1]

    def body(p_ref, dinv_ref, dinv2_ref):
        deg = p_ref[0, :, 0:1] + p_ref[1, :, 0:1] + 1.0
        dinv_ref[...] = lax.rsqrt(deg)
        dinv2_ref[...] = 1.0 / deg

    return pl.pallas_call(
        body,
        out_shape=[
            jax.ShapeDtypeStruct((n, 1), jnp.float32),
            jax.ShapeDtypeStruct((n, 1), jnp.float32),
        ],
    )(parts)


def _tc_ee(edge_attr, Wes, bes):
    e, de = edge_attr.shape
    nl, _, d = Wes.shape
    be_blk = 2000
    grid = (nl, e // be_blk)

    def body(ea_ref, w_ref, b_ref, out_ref):
        acc = jnp.dot(ea_ref[...], w_ref[0], preferred_element_type=jnp.float32)
        out_ref[0] = acc + b_ref[0]

    return pl.pallas_call(
        body,
        grid=grid,
        in_specs=[
            pl.BlockSpec((be_blk, de), lambda l, i: (i, 0)),
            pl.BlockSpec((1, de, d), lambda l, i: (l, 0, 0)),
            pl.BlockSpec((1, 1, d), lambda l, i: (l, 0, 0)),
        ],
        out_specs=pl.BlockSpec((1, be_blk, d), lambda l, i: (l, i, 0)),
        out_shape=jax.ShapeDtypeStruct((nl, e, d), jnp.float32),
    )(edge_attr, Wes, bes.reshape(nl, 1, d))


def _tc_matmul(h, W, b):
    n, d = h.shape

    def body(h_ref, w_ref, b_ref, out_ref):
        out_ref[...] = (
            jnp.dot(h_ref[...], w_ref[...], preferred_element_type=jnp.float32)
            + b_ref[...]
        )

    return pl.pallas_call(
        body,
        out_shape=jax.ShapeDtypeStruct((n, d), jnp.float32),
    )(h, W, b.reshape(1, d))


def _tc_combine(parts, h_lin, root, dinv2):
    n, d = h_lin.shape

    def body(p_ref, h_ref, r_ref, d2_ref, out_ref):
        aggr = p_ref[0] + p_ref[1]
        self_term = jnp.maximum(h_ref[...] + r_ref[...], 0.0) * d2_ref[...]
        out_ref[...] = jnp.maximum(aggr + self_term, 0.0)

    return pl.pallas_call(
        body,
        out_shape=jax.ShapeDtypeStruct((n, d), jnp.float32),
    )(parts, h_lin, root.reshape(1, d), dinv2)


def _tc_final(hs, gamma, beta, Wout, bout):
    n, d = hs[0].shape
    nseg = len(hs)

    def body(x0, x1, x2, x3, g_ref, be_ref, w_ref, bo_ref, out_ref):
        acc = jnp.zeros((n, d), jnp.float32)
        for i, xr in enumerate((x0, x1, x2, x3)):
            xv = xr[...]
            m = jnp.mean(xv, axis=0, keepdims=True)
            ctr = xv - m
            var = jnp.mean(ctr * ctr, axis=0, keepdims=True)
            xn = ctr * lax.rsqrt(var + 1e-5) * g_ref[i] + be_ref[i]
            acc = acc + jnp.dot(xn, w_ref[i], preferred_element_type=jnp.float32)
        out_ref[...] = acc + bo_ref[...]

    return pl.pallas_call(
        body,
        out_shape=jax.ShapeDtypeStruct((n, d), jnp.float32),
    )(*hs, gamma.reshape(nseg, d), beta.reshape(nseg, d),
      Wout.reshape(nseg, d, d), bout.reshape(1, d))


def kernel(x, edge_index, edge_attr, Ws, bs, Wes, bes, roots, gamma, beta,
           Wout, bout):
    n, d = x.shape
    nl = Ws.shape[0]
    row = edge_index[0]
    col = edge_index[1]

    deg_parts = _sc_degree(row, n).reshape(NC, n, LANES)
    dinv, dinv2 = _tc_deg_finish(deg_parts)
    norm = _sc_norm(row, col, dinv.reshape(n))
    ee3 = _tc_ee(edge_attr, Wes, bes)

    h = x
    hs = [x]
    for l in range(nl):
        h_lin = _tc_matmul(h, Ws[l], bs[l])
        parts = _sc_message(h_lin, ee3, l, row, col, norm, n, d)
        h = _tc_combine(parts.reshape(NC, n, d), h_lin, roots[l], dinv2)
        hs.append(h)

    return _tc_final(hs, gamma, beta, Wout, bout)


# speedup vs baseline: 5.3068x; 1.6178x over previous
import dataclasses
import functools

import jax
import jax.numpy as jnp
from jax import lax
from jax.experimental import pallas as pl
from jax.experimental.pallas import tpu as pltpu
from jax.experimental.pallas import tpu_sc as plsc

NC = 2
NS = 16
NW = NC * NS
LANES = 16

_MESH = plsc.VectorSubcoreMesh(
    core_axis_name="c", subcore_axis_name="s", num_cores=NC, num_subcores=NS
)

_SC_PARAMS = pltpu.CompilerParams()
if "needs_layout_passes" in pltpu.CompilerParams.__dataclass_fields__:
    _SC_PARAMS = dataclasses.replace(_SC_PARAMS, needs_layout_passes=False)

_SC_LINEAR = dataclasses.replace(
    pltpu.CompilerParams(), use_tc_tiling_on_sc=False
)


def _scatter_add(src_ref, dst_ref, idx_ref):
    pltpu.sync_copy(src_ref, dst_ref.at[idx_ref], add=True)


def _sc_degree(row, n):
    e = row.shape[0]
    epw = e // NW
    b = 80
    nrounds = epw // b
    nps = n // NS
    zr = 125
    assert nrounds * b == epw and nps % zr == 0

    @functools.partial(
        pl.kernel,
        out_type=jax.ShapeDtypeStruct((NC, NS, nps, LANES), jnp.float32),
        mesh=_MESH,
        compiler_params=_SC_LINEAR,
        scratch_types=[
            pltpu.VMEM((b,), jnp.int32),
            pltpu.VMEM((b, LANES), jnp.float32),
            pltpu.VMEM((zr, LANES), jnp.float32),
            pltpu.VMEM_SHARED((n, LANES), jnp.float32),
        ],
    )
    def k(row_hbm, out_hbm, idx_v, ones_v, z_v, acc_sh):
        cid = lax.axis_index("c")
        sid = lax.axis_index("s")
        wid = cid * NS + sid

        @pl.loop(0, b)
        def _(i):
            ones_v[i, :] = jnp.ones((LANES,), jnp.float32)

        @pl.loop(0, zr)
        def _(i):
            z_v[i, :] = jnp.zeros((LANES,), jnp.float32)

        @pl.loop(0, nps // zr)
        def _(t):
            pltpu.sync_copy(z_v, acc_sh.at[pl.ds(sid * nps + t * zr, zr)])

        plsc.subcore_barrier()

        base = wid * epw

        @pl.loop(0, nrounds)
        def _(r):
            pltpu.sync_copy(row_hbm.at[pl.ds(base + r * b, b)], idx_v)
            _scatter_add(ones_v, acc_sh, idx_v)

        plsc.subcore_barrier()
        pltpu.sync_copy(
            acc_sh.at[pl.ds(sid * nps, nps)],
            out_hbm.at[cid, sid],
        )

    return k(row)


def _sc_norm(row, col, dinv):
    e = row.shape[0]
    n = dinv.shape[0]
    epw = e // NW
    b = 80
    nrounds = epw // b

    @functools.partial(
        pl.kernel,
        out_type=jax.ShapeDtypeStruct((e,), jnp.float32),
        mesh=_MESH,
        compiler_params=_SC_PARAMS,
        scratch_types=[
            pltpu.VMEM((n,), jnp.float32),
            pltpu.VMEM((b,), jnp.int32),
            pltpu.VMEM((b,), jnp.int32),
            pltpu.VMEM((b,), jnp.float32),
        ],
    )
    def k(row_hbm, col_hbm, dinv_hbm, out_hbm, dinv_v, row_v, col_v, norm_v):
        cid = lax.axis_index("c")
        sid = lax.axis_index("s")
        wid = cid * NS + sid
        base = wid * epw
        pltpu.sync_copy(dinv_hbm, dinv_v)

        @pl.loop(0, nrounds)
        def _(r):
            pltpu.sync_copy(row_hbm.at[pl.ds(base + r * b, b)], row_v)
            pltpu.sync_copy(col_hbm.at[pl.ds(base + r * b, b)], col_v)

            @pl.loop(0, b // LANES)
            def _(j):
                ir = row_v[pl.ds(j * LANES, LANES)]
                ic = col_v[pl.ds(j * LANES, LANES)]
                gr = plsc.load_gather(dinv_v, [ir])
                gc = plsc.load_gather(dinv_v, [ic])
                norm_v[pl.ds(j * LANES, LANES)] = gr * gc

            pltpu.sync_copy(norm_v, out_hbm.at[pl.ds(base + r * b, b)])

    return k(row, col, dinv)


def _sc_message(h_lin, ee3, l, row1d, col2d, norm2d, n, d):
    b = col2d.shape[2]
    rpw = col2d.shape[1]
    e = col2d.shape[0] * rpw * b
    epw = e // NW
    nps = n // NS
    zr = 25
    nchunks = d // LANES
    assert rpw * b == epw and (nps % zr) == 0 and rpw % 2 == 0 and zr <= b

    ngroups = []
    off = 0
    while off + LANES <= b:
        ngroups.append((off, 0))
        off += LANES
    if off < b:
        ngroups.append((b - LANES, LANES - (b - off)))

    @functools.partial(
        pl.kernel,
        out_type=jax.ShapeDtypeStruct((NC, NS, nps, d), jnp.float32),
        mesh=_MESH,
        scratch_types=[
            pltpu.VMEM((epw,), jnp.int32),
            pltpu.VMEM((2, b), jnp.int32),
            pltpu.VMEM((2, b), jnp.float32),
            pltpu.VMEM((2, b, d), jnp.float32),
            pltpu.VMEM((2, b, d), jnp.float32),
            pltpu.VMEM((2, b, d), jnp.float32),
            pltpu.VMEM_SHARED((n, d), jnp.float32),
            pltpu.SemaphoreType.DMA,
            pltpu.SemaphoreType.DMA,
            pltpu.SemaphoreType.DMA,
            pltpu.SemaphoreType.DMA,
            pltpu.SemaphoreType.DMA,
            pltpu.SemaphoreType.DMA,
            pltpu.SemaphoreType.DMA,
            pltpu.SemaphoreType.DMA,
            pltpu.SemaphoreType.DMA,
            pltpu.SemaphoreType.DMA,
        ],
    )
    def k(h_hbm, ee_hbm, row_hbm, col_hbm, norm_hbm, out_hbm,
          row_s, col_r, nrm_r, g_v, ee_v, m_v, acc_sh,
          sg0, sg1, se0, se1, ss0, ss1, sn0, sn1, sc0, sc1):
        cid = lax.axis_index("c")
        sid = lax.axis_index("s")
        wid = cid * NS + sid
        r0 = wid * rpw
        sg = (sg0, sg1)
        se = (se0, se1)
        ss = (ss0, ss1)
        sn = (sn0, sn1)
        sc = (sc0, sc1)

        pltpu.sync_copy(row_hbm.at[pl.ds(wid * epw, epw)], row_s)

        @pl.loop(0, zr)
        def _(i):
            for c in range(nchunks):
                m_v[0, i, pl.ds(c * LANES, LANES)] = jnp.zeros(
                    (LANES,), jnp.float32)

        @pl.loop(0, nps // zr)
        def _(t):
            pltpu.sync_copy(m_v.at[0, pl.ds(0, zr)],
                            acc_sh.at[pl.ds(sid * nps + t * zr, zr)])

        plsc.subcore_barrier()

        def issue_in(r, k_):
            pltpu.async_copy(ee_hbm.at[l, pl.ds((r0 + r) * b, b)],
                             ee_v.at[k_], se[k_])
            pltpu.async_copy(norm_hbm.at[wid, r], nrm_r.at[k_], sn[k_])
            pltpu.async_copy(h_hbm.at[row_s.at[pl.ds(r * b, b)]],
                             g_v.at[k_], sg[k_])

        def wait_in(r, k_):
            pltpu.make_async_copy(ee_hbm.at[l, pl.ds((r0 + r) * b, b)],
                                  ee_v.at[k_], se[k_]).wait()
            pltpu.make_async_copy(norm_hbm.at[wid, r], nrm_r.at[k_],
                                  sn[k_]).wait()
            pltpu.make_async_copy(h_hbm.at[row_s.at[pl.ds(r * b, b)]],
                                  g_v.at[k_], sg[k_]).wait()

        def issue_col(r, k_):
            pltpu.async_copy(col_hbm.at[wid, r], col_r.at[k_], sc[k_])

        def wait_col(r, k_):
            pltpu.make_async_copy(col_hbm.at[wid, r], col_r.at[k_],
                                  sc[k_]).wait()

        def issue_sc(r, k_):
            pltpu.async_copy(m_v.at[k_], acc_sh.at[col_r.at[k_]], ss[k_],
                             add=True)

        def wait_sc(r, k_):
            pltpu.make_async_copy(m_v.at[k_], acc_sh.at[col_r.at[k_]],
                                  ss[k_]).wait()

        def compute(r, k_):
            for off, j0 in ngroups:
                nv = nrm_r[k_, pl.ds(off, LANES)]
                for jj in range(j0, LANES):
                    j = off + jj
                    nj = nv[jj]
                    for c in range(nchunks):
                        sl = pl.ds(c * LANES, LANES)
                        v = g_v[k_, j, sl] + ee_v[k_, j, sl]
                        m_v[k_, j, sl] = jnp.maximum(v, 0.0) * nj

        issue_in(0, 0)
        issue_in(1, 1)

        @pl.loop(0, rpw // 2)
        def _(j):
            for k_ in range(2):
                r = 2 * j + k_

                @pl.when(j > 0)
                def _():
                    wait_sc(r - 2, k_)

                issue_col(r, k_)
                wait_in(r, k_)
                compute(r, k_)
                wait_col(r, k_)
                issue_sc(r, k_)

                @pl.when(r + 2 < rpw)
                def _():
                    issue_in(r + 2, k_)

        wait_sc(rpw - 2, 0)
        wait_sc(rpw - 1, 1)

        plsc.subcore_barrier()
        pltpu.sync_copy(
            acc_sh.at[pl.ds(sid * nps, nps)],
            out_hbm.at[cid, sid],
        )

    return k(h_lin, ee3, row1d, col2d, norm2d)


def _tc_deg_finish(parts):
    n = parts.shape[1]

    def body(p_ref, dinv_ref, dinv2_ref):
        deg = p_ref[0, :, 0:1] + p_ref[1, :, 0:1] + 1.0
        dinv_ref[...] = lax.rsqrt(deg)
        dinv2_ref[...] = 1.0 / deg

    return pl.pallas_call(
        body,
        out_shape=[
            jax.ShapeDtypeStruct((n, 1), jnp.float32),
            jax.ShapeDtypeStruct((n, 1), jnp.float32),
        ],
    )(parts)


def _tc_ee(edge_attr, Wes, bes):
    e, de = edge_attr.shape
    nl, _, d = Wes.shape
    be_blk = 2000
    grid = (nl, e // be_blk)

    def body(ea_ref, w_ref, b_ref, out_ref):
        acc = jnp.dot(ea_ref[...], w_ref[0], preferred_element_type=jnp.float32)
        out_ref[0] = acc + b_ref[0]

    return pl.pallas_call(
        body,
        grid=grid,
        in_specs=[
            pl.BlockSpec((be_blk, de), lambda l, i: (i, 0)),
            pl.BlockSpec((1, de, d), lambda l, i: (l, 0, 0)),
            pl.BlockSpec((1, 1, d), lambda l, i: (l, 0, 0)),
        ],
        out_specs=pl.BlockSpec((1, be_blk, d), lambda l, i: (l, i, 0)),
        out_shape=jax.ShapeDtypeStruct((nl, e, d), jnp.float32),
    )(edge_attr, Wes, bes.reshape(nl, 1, d))


def _tc_matmul(h, W, b):
    n, d = h.shape

    def body(h_ref, w_ref, b_ref, out_ref):
        out_ref[...] = (
            jnp.dot(h_ref[...], w_ref[...], preferred_element_type=jnp.float32)
            + b_ref[...]
        )

    return pl.pallas_call(
        body,
        out_shape=jax.ShapeDtypeStruct((n, d), jnp.float32),
    )(h, W, b.reshape(1, d))


def _tc_combine(parts, h_lin, root, dinv2):
    n, d = h_lin.shape

    def body(p_ref, h_ref, r_ref, d2_ref, out_ref):
        aggr = p_ref[0] + p_ref[1]
        self_term = jnp.maximum(h_ref[...] + r_ref[...], 0.0) * d2_ref[...]
        out_ref[...] = jnp.maximum(aggr + self_term, 0.0)

    return pl.pallas_call(
        body,
        out_shape=jax.ShapeDtypeStruct((n, d), jnp.float32),
    )(parts, h_lin, root.reshape(1, d), dinv2)


def _tc_final(hs, gamma, beta, Wout, bout):
    n, d = hs[0].shape
    nseg = len(hs)

    def body(x0, x1, x2, x3, g_ref, be_ref, w_ref, bo_ref, out_ref):
        acc = jnp.zeros((n, d), jnp.float32)
        for i, xr in enumerate((x0, x1, x2, x3)):
            xv = xr[...]
            m = jnp.mean(xv, axis=0, keepdims=True)
            ctr = xv - m
            var = jnp.mean(ctr * ctr, axis=0, keepdims=True)
            xn = ctr * lax.rsqrt(var + 1e-5) * g_ref[i] + be_ref[i]
            acc = acc + jnp.dot(xn, w_ref[i], preferred_element_type=jnp.float32)
        out_ref[...] = acc + bo_ref[...]

    return pl.pallas_call(
        body,
        out_shape=jax.ShapeDtypeStruct((n, d), jnp.float32),
    )(*hs, gamma.reshape(nseg, d), beta.reshape(nseg, d),
      Wout.reshape(nseg, d, d), bout.reshape(1, d))


def kernel(x, edge_index, edge_attr, Ws, bs, Wes, bes, roots, gamma, beta,
           Wout, bout):
    n, d = x.shape
    nl = Ws.shape[0]
    row = edge_index[0]
    col = edge_index[1]

    deg_parts = _sc_degree(row, n).reshape(NC, n, LANES)
    dinv, dinv2 = _tc_deg_finish(deg_parts)
    norm = _sc_norm(row, col, dinv.reshape(n))
    ee3 = _tc_ee(edge_attr, Wes, bes)

    e = row.shape[0]
    b = 40
    rpw = e // (NW * b)
    col3 = col.reshape(NW, rpw, b)
    norm3 = norm.reshape(NW, rpw, b)

    h = x
    hs = [x]
    for l in range(nl):
        h_lin = _tc_matmul(h, Ws[l], bs[l])
        parts = _sc_message(h_lin, ee3, l, row, col3, norm3, n, d)
        h = _tc_combine(parts.reshape(NC, n, d), h_lin, roots[l], dinv2)
        hs.append(h)

    return _tc_final(hs, gamma, beta, Wout, bout)


# speedup vs baseline: 5.3389x; 1.0061x over previous
import dataclasses
import functools

import jax
import jax.numpy as jnp
from jax import lax
from jax.experimental import pallas as pl
from jax.experimental.pallas import tpu as pltpu
from jax.experimental.pallas import tpu_sc as plsc

NC = 2
NS = 16
NW = NC * NS
LANES = 16

_MESH = plsc.VectorSubcoreMesh(
    core_axis_name="c", subcore_axis_name="s", num_cores=NC, num_subcores=NS
)

_SC_PARAMS = pltpu.CompilerParams()
if "needs_layout_passes" in pltpu.CompilerParams.__dataclass_fields__:
    _SC_PARAMS = dataclasses.replace(_SC_PARAMS, needs_layout_passes=False)

_SC_LINEAR = dataclasses.replace(
    pltpu.CompilerParams(), use_tc_tiling_on_sc=False
)


def _scatter_add(src_ref, dst_ref, idx_ref):
    pltpu.sync_copy(src_ref, dst_ref.at[idx_ref], add=True)


def _sc_degree(row, n):
    e = row.shape[0]
    epw = e // NW
    b = 80
    nrounds = epw // b
    nps = n // NS
    zr = 125
    assert nrounds * b == epw and nps % zr == 0

    @functools.partial(
        pl.kernel,
        out_type=jax.ShapeDtypeStruct((NC, NS, nps, LANES), jnp.float32),
        mesh=_MESH,
        compiler_params=_SC_LINEAR,
        scratch_types=[
            pltpu.VMEM((b,), jnp.int32),
            pltpu.VMEM((b, LANES), jnp.float32),
            pltpu.VMEM((zr, LANES), jnp.float32),
            pltpu.VMEM_SHARED((n, LANES), jnp.float32),
        ],
    )
    def k(row_hbm, out_hbm, idx_v, ones_v, z_v, acc_sh):
        cid = lax.axis_index("c")
        sid = lax.axis_index("s")
        wid = cid * NS + sid

        @pl.loop(0, b)
        def _(i):
            ones_v[i, :] = jnp.ones((LANES,), jnp.float32)

        @pl.loop(0, zr)
        def _(i):
            z_v[i, :] = jnp.zeros((LANES,), jnp.float32)

        @pl.loop(0, nps // zr)
        def _(t):
            pltpu.sync_copy(z_v, acc_sh.at[pl.ds(sid * nps + t * zr, zr)])

        plsc.subcore_barrier()

        base = wid * epw

        @pl.loop(0, nrounds)
        def _(r):
            pltpu.sync_copy(row_hbm.at[pl.ds(base + r * b, b)], idx_v)
            _scatter_add(ones_v, acc_sh, idx_v)

        plsc.subcore_barrier()
        pltpu.sync_copy(
            acc_sh.at[pl.ds(sid * nps, nps)],
            out_hbm.at[cid, sid],
        )

    return k(row)


def _sc_norm(row, col, dinv):
    e = row.shape[0]
    n = dinv.shape[0]
    epw = e // NW
    b = 80
    nrounds = epw // b

    @functools.partial(
        pl.kernel,
        out_type=jax.ShapeDtypeStruct((e,), jnp.float32),
        mesh=_MESH,
        compiler_params=_SC_PARAMS,
        scratch_types=[
            pltpu.VMEM((n,), jnp.float32),
            pltpu.VMEM((b,), jnp.int32),
            pltpu.VMEM((b,), jnp.int32),
            pltpu.VMEM((b,), jnp.float32),
        ],
    )
    def k(row_hbm, col_hbm, dinv_hbm, out_hbm, dinv_v, row_v, col_v, norm_v):
        cid = lax.axis_index("c")
        sid = lax.axis_index("s")
        wid = cid * NS + sid
        base = wid * epw
        pltpu.sync_copy(dinv_hbm, dinv_v)

        @pl.loop(0, nrounds)
        def _(r):
            pltpu.sync_copy(row_hbm.at[pl.ds(base + r * b, b)], row_v)
            pltpu.sync_copy(col_hbm.at[pl.ds(base + r * b, b)], col_v)

            @pl.loop(0, b // LANES)
            def _(j):
                ir = row_v[pl.ds(j * LANES, LANES)]
                ic = col_v[pl.ds(j * LANES, LANES)]
                gr = plsc.load_gather(dinv_v, [ir])
                gc = plsc.load_gather(dinv_v, [ic])
                norm_v[pl.ds(j * LANES, LANES)] = gr * gc

            pltpu.sync_copy(norm_v, out_hbm.at[pl.ds(base + r * b, b)])

    return k(row, col, dinv)


def _sc_message(h_lin, ee3, l, row1d, col2d, norm2d, n, d):
    b = col2d.shape[2]
    rpw = col2d.shape[1]
    e = col2d.shape[0] * rpw * b
    epw = e // NW
    nps = n // NS
    zr = 25
    nchunks = d // LANES
    assert rpw * b == epw and (nps % zr) == 0 and rpw % 2 == 0 and zr <= b

    ngroups = []
    off = 0
    while off + LANES <= b:
        ngroups.append((off, 0))
        off += LANES
    if off < b:
        ngroups.append((b - LANES, LANES - (b - off)))

    @functools.partial(
        pl.kernel,
        out_type=jax.ShapeDtypeStruct((NC, NS, nps, d), jnp.float32),
        mesh=_MESH,
        scratch_types=[
            pltpu.VMEM((epw,), jnp.int32),
            pltpu.VMEM((2, b), jnp.int32),
            pltpu.VMEM((2, b), jnp.float32),
            pltpu.VMEM((2, b, d), jnp.float32),
            pltpu.VMEM((2, b, d), jnp.float32),
            pltpu.VMEM((2, b, d), jnp.float32),
            pltpu.VMEM_SHARED((n, d), jnp.float32),
            pltpu.SemaphoreType.DMA,
            pltpu.SemaphoreType.DMA,
            pltpu.SemaphoreType.DMA,
            pltpu.SemaphoreType.DMA,
            pltpu.SemaphoreType.DMA,
            pltpu.SemaphoreType.DMA,
            pltpu.SemaphoreType.DMA,
            pltpu.SemaphoreType.DMA,
            pltpu.SemaphoreType.DMA,
            pltpu.SemaphoreType.DMA,
        ],
    )
    def k(h_hbm, ee_hbm, row_hbm, col_hbm, norm_hbm, out_hbm,
          row_s, col_r, nrm_r, g_v, ee_v, m_v, acc_sh,
          sg0, sg1, se0, se1, ss0, ss1, sn0, sn1, sc0, sc1):
        cid = lax.axis_index("c")
        sid = lax.axis_index("s")
        wid = cid * NS + sid
        r0 = wid * rpw
        sg = (sg0, sg1)
        se = (se0, se1)
        ss = (ss0, ss1)
        sn = (sn0, sn1)
        sc = (sc0, sc1)

        pltpu.sync_copy(row_hbm.at[pl.ds(wid * epw, epw)], row_s)

        @pl.loop(0, zr)
        def _(i):
            for c in range(nchunks):
                m_v[0, i, pl.ds(c * LANES, LANES)] = jnp.zeros(
                    (LANES,), jnp.float32)

        @pl.loop(0, nps // zr)
        def _(t):
            pltpu.sync_copy(m_v.at[0, pl.ds(0, zr)],
                            acc_sh.at[pl.ds(sid * nps + t * zr, zr)])

        plsc.subcore_barrier()

        def issue_in(r, k_):
            pltpu.async_copy(ee_hbm.at[l, pl.ds((r0 + r) * b, b)],
                             ee_v.at[k_], se[k_])
            pltpu.async_copy(norm_hbm.at[wid, r], nrm_r.at[k_], sn[k_])
            pltpu.async_copy(h_hbm.at[row_s.at[pl.ds(r * b, b)]],
                             g_v.at[k_], sg[k_])

        def wait_in(r, k_):
            pltpu.make_async_copy(ee_hbm.at[l, pl.ds((r0 + r) * b, b)],
                                  ee_v.at[k_], se[k_]).wait()
            pltpu.make_async_copy(norm_hbm.at[wid, r], nrm_r.at[k_],
                                  sn[k_]).wait()
            pltpu.make_async_copy(h_hbm.at[row_s.at[pl.ds(r * b, b)]],
                                  g_v.at[k_], sg[k_]).wait()

        def issue_col(r, k_):
            pltpu.async_copy(col_hbm.at[wid, r], col_r.at[k_], sc[k_])

        def wait_col(r, k_):
            pltpu.make_async_copy(col_hbm.at[wid, r], col_r.at[k_],
                                  sc[k_]).wait()

        def issue_sc(r, k_):
            pltpu.async_copy(m_v.at[k_], acc_sh.at[col_r.at[k_]], ss[k_],
                             add=True)

        def wait_sc(r, k_):
            pltpu.make_async_copy(m_v.at[k_], acc_sh.at[col_r.at[k_]],
                                  ss[k_]).wait()

        def compute(r, k_):
            for off, j0 in ngroups:
                nv = nrm_r[k_, pl.ds(off, LANES)]
                for jj in range(j0, LANES):
                    j = off + jj
                    nj = nv[jj]
                    for c in range(nchunks):
                        sl = pl.ds(c * LANES, LANES)
                        v = g_v[k_, j, sl] + ee_v[k_, j, sl]
                        m_v[k_, j, sl] = jnp.maximum(v, 0.0) * nj

        issue_in(0, 0)
        issue_in(1, 1)

        @pl.loop(0, rpw // 2)
        def _(j):
            for k_ in range(2):
                r = 2 * j + k_

                @pl.when(j > 0)
                def _():
                    wait_sc(r - 2, k_)

                issue_col(r, k_)
                wait_in(r, k_)
                compute(r, k_)
                wait_col(r, k_)
                issue_sc(r, k_)

                @pl.when(r + 2 < rpw)
                def _():
                    issue_in(r + 2, k_)

        wait_sc(rpw - 2, 0)
        wait_sc(rpw - 1, 1)

        plsc.subcore_barrier()
        pltpu.sync_copy(
            acc_sh.at[pl.ds(sid * nps, nps)],
            out_hbm.at[cid, sid],
        )

    return k(h_lin, ee3, row1d, col2d, norm2d)


def _tc_deg_finish(parts):
    n = parts.shape[1]

    def body(p_ref, dinv_ref, dinv2_ref):
        deg = p_ref[0, :, 0:1] + p_ref[1, :, 0:1] + 1.0
        dinv_ref[...] = lax.rsqrt(deg)
        dinv2_ref[...] = 1.0 / deg

    return pl.pallas_call(
        body,
        out_shape=[
            jax.ShapeDtypeStruct((n, 1), jnp.float32),
            jax.ShapeDtypeStruct((n, 1), jnp.float32),
        ],
    )(parts)


def _tc_ee(edge_attr, Wes, bes):
    e, de = edge_attr.shape
    nl, _, d = Wes.shape
    be_blk = 2000
    grid = (nl, e // be_blk)

    def body(ea_ref, w_ref, b_ref, out_ref):
        acc = jnp.dot(ea_ref[...], w_ref[0], preferred_element_type=jnp.float32)
        out_ref[0] = acc + b_ref[0]

    return pl.pallas_call(
        body,
        grid=grid,
        in_specs=[
            pl.BlockSpec((be_blk, de), lambda l, i: (i, 0)),
            pl.BlockSpec((1, de, d), lambda l, i: (l, 0, 0)),
            pl.BlockSpec((1, 1, d), lambda l, i: (l, 0, 0)),
        ],
        out_specs=pl.BlockSpec((1, be_blk, d), lambda l, i: (l, i, 0)),
        out_shape=jax.ShapeDtypeStruct((nl, e, d), jnp.float32),
    )(edge_attr, Wes, bes.reshape(nl, 1, d))


def _tc_matmul(h, W, b):
    n, d = h.shape

    def body(h_ref, w_ref, b_ref, out_ref):
        out_ref[...] = (
            jnp.dot(h_ref[...], w_ref[...], preferred_element_type=jnp.float32)
            + b_ref[...]
        )

    return pl.pallas_call(
        body,
        out_shape=jax.ShapeDtypeStruct((n, d), jnp.float32),
    )(h, W, b.reshape(1, d))


def _tc_combine_matmul(parts, h_lin, root, dinv2, W, b):
    n, d = h_lin.shape

    def body(p_ref, h_ref, r_ref, d2_ref, w_ref, b_ref, out_ref, lin_ref):
        aggr = p_ref[0] + p_ref[1]
        self_term = jnp.maximum(h_ref[...] + r_ref[...], 0.0) * d2_ref[...]
        h = jnp.maximum(aggr + self_term, 0.0)
        out_ref[...] = h
        lin_ref[...] = (
            jnp.dot(h, w_ref[...], preferred_element_type=jnp.float32)
            + b_ref[...]
        )

    return pl.pallas_call(
        body,
        out_shape=[
            jax.ShapeDtypeStruct((n, d), jnp.float32),
            jax.ShapeDtypeStruct((n, d), jnp.float32),
        ],
    )(parts, h_lin, root.reshape(1, d), dinv2, W, b.reshape(1, d))


def _tc_final(hs, parts, h_lin, root, dinv2, gamma, beta, Wout, bout):
    n, d = hs[0].shape
    nseg = len(hs) + 1

    def body(x0, x1, x2, p_ref, h_ref, r_ref, d2_ref,
             g_ref, be_ref, w_ref, bo_ref, out_ref):
        aggr = p_ref[0] + p_ref[1]
        self_term = jnp.maximum(h_ref[...] + r_ref[...], 0.0) * d2_ref[...]
        h3 = jnp.maximum(aggr + self_term, 0.0)
        acc = jnp.zeros((n, d), jnp.float32)
        for i, xv in enumerate((x0[...], x1[...], x2[...], h3)):
            m = jnp.mean(xv, axis=0, keepdims=True)
            ctr = xv - m
            var = jnp.mean(ctr * ctr, axis=0, keepdims=True)
            xn = ctr * lax.rsqrt(var + 1e-5) * g_ref[i] + be_ref[i]
            acc = acc + jnp.dot(xn, w_ref[i], preferred_element_type=jnp.float32)
        out_ref[...] = acc + bo_ref[...]

    return pl.pallas_call(
        body,
        out_shape=jax.ShapeDtypeStruct((n, d), jnp.float32),
    )(*hs, parts, h_lin, root.reshape(1, d), dinv2,
      gamma.reshape(nseg, d), beta.reshape(nseg, d),
      Wout.reshape(nseg, d, d), bout.reshape(1, d))


def kernel(x, edge_index, edge_attr, Ws, bs, Wes, bes, roots, gamma, beta,
           Wout, bout):
    n, d = x.shape
    nl = Ws.shape[0]
    row = edge_index[0]
    col = edge_index[1]

    deg_parts = _sc_degree(row, n).reshape(NC, n, LANES)
    dinv, dinv2 = _tc_deg_finish(deg_parts)
    norm = _sc_norm(row, col, dinv.reshape(n))
    ee3 = _tc_ee(edge_attr, Wes, bes)

    e = row.shape[0]
    b = 40
    rpw = e // (NW * b)
    col3 = col.reshape(NW, rpw, b)
    norm3 = norm.reshape(NW, rpw, b)

    hs = [x]
    h_lin = _tc_matmul(x, Ws[0], bs[0])
    for l in range(nl - 1):
        parts = _sc_message(h_lin, ee3, l, row, col3, norm3, n, d)
        h, h_lin = _tc_combine_matmul(parts.reshape(NC, n, d), h_lin,
                                      roots[l], dinv2, Ws[l + 1], bs[l + 1])
        hs.append(h)

    parts = _sc_message(h_lin, ee3, nl - 1, row, col3, norm3, n, d)
    return _tc_final(hs, parts.reshape(NC, n, d), h_lin, roots[nl - 1],
                     dinv2, gamma, beta, Wout, bout)
